# baseline (speedup 1.0000x reference)
import jax
import jax.numpy as jnp
from jax import lax
from jax.experimental import pallas as pl
from jax.experimental.pallas import tpu as pltpu
from jax.experimental.pallas import tpu_sc as plsc

T = 8192
DM = 1024
E = 64
NA = 32
TILE = 1024
NT = T // TILE
_BIG = 2**30

_CONTRACT_MINOR = (((1,), (1,)), ((), ()))

_GATHER_DN = lax.GatherDimensionNumbers(
    offset_dims=(), collapsed_slice_dims=(0,), start_index_map=(0,))


def _lane_gather(x, idx):
    return lax.gather(x, idx[:, None], _GATHER_DN, slice_sizes=(1,),
                      mode=lax.GatherScatterMode.PROMISE_IN_BOUNDS)


def _logits_body(obs_ref, wgt_ref, bg_ref, pv_ref, pi_ref):
    i = pl.program_id(0)
    logits_t = lax.dot_general(
        wgt_ref[...], obs_ref[...], _CONTRACT_MINOR,
        preferred_element_type=jnp.float32) + bg_ref[...].T
    m = jnp.max(logits_t)
    erow = lax.broadcasted_iota(jnp.int32, (E, TILE), 0)
    tcol = lax.broadcasted_iota(jnp.int32, (E, TILE), 1)
    flat = (i * TILE + tcol) * E + erow
    idx = jnp.min(jnp.where(logits_t == m, flat, _BIG))
    pv_ref[i] = m
    pi_ref[i] = idx


def _sc_finalize(pv_hbm, pi_hbm, eidx_hbm, pv_v, pi_v, e_v):
    c = lax.axis_index("c")
    s = lax.axis_index("s")

    @pl.when((c == 0) & (s == 0))
    def _():
        pltpu.sync_copy(pv_hbm, pv_v)
        pltpu.sync_copy(pi_hbm, pi_v)
        pv = pv_v[...]
        pi = pi_v[...]
        lane = lax.iota(jnp.int32, 16)
        for k in (8, 4, 2, 1):
            perm = lane ^ k
            pvr = _lane_gather(pv, perm)
            pir = _lane_gather(pi, perm)
            better = (pvr > pv) | ((pvr == pv) & (pir < pi))
            pv = jnp.where(better, pvr, pv)
            pi = jnp.where(better, pir, pi)
        e_v[...] = pi % E
        pltpu.sync_copy(e_v, eidx_hbm)


def _expert_body(eidx_ref, obs_ref, wet_ref, bet_ref, out_ref):
    e = eidx_ref[0]
    cols = lax.broadcasted_iota(jnp.int32, (NA, E), 1)
    b = jnp.sum(jnp.where(cols == e, bet_ref[...], 0.0),
                axis=1, keepdims=True)
    out_ref[...] = lax.dot_general(
        wet_ref[0], obs_ref[...], _CONTRACT_MINOR,
        preferred_element_type=jnp.float32) + b


def kernel(context, obs, Wg, bg, We, be):
    del context
    wgt = Wg.T
    wet = jnp.swapaxes(We, 1, 2)
    bet = be.T
    bg2 = bg.reshape(1, E)

    pv, pi = pl.pallas_call(
        _logits_body,
        grid=(NT,),
        in_specs=[
            pl.BlockSpec((TILE, DM), lambda i: (i, 0)),
            pl.BlockSpec((E, DM), lambda i: (0, 0)),
            pl.BlockSpec((1, E), lambda i: (0, 0)),
        ],
        out_specs=[
            pl.BlockSpec(memory_space=pltpu.SMEM),
            pl.BlockSpec(memory_space=pltpu.SMEM),
        ],
        out_shape=[
            jax.ShapeDtypeStruct((NT,), jnp.float32),
            jax.ShapeDtypeStruct((NT,), jnp.int32),
        ],
        compiler_params=pltpu.CompilerParams(
            dimension_semantics=("arbitrary",),
        ),
    )(obs, wgt, bg2)

    pv16 = jnp.full((16,), -jnp.inf, jnp.float32).at[:NT].set(pv)
    pi16 = jnp.full((16,), _BIG, jnp.int32).at[:NT].set(pi)

    mesh = plsc.VectorSubcoreMesh(core_axis_name="c", subcore_axis_name="s")
    eidx = pl.kernel(
        _sc_finalize,
        mesh=mesh,
        out_type=jax.ShapeDtypeStruct((16,), jnp.int32),
        scratch_types=[
            pltpu.VMEM((16,), jnp.float32),
            pltpu.VMEM((16,), jnp.int32),
            pltpu.VMEM((16,), jnp.int32),
        ],
    )(pv16, pi16)

    grid_spec = pltpu.PrefetchScalarGridSpec(
        num_scalar_prefetch=1,
        grid=(NT,),
        in_specs=[
            pl.BlockSpec((TILE, DM), lambda i, e: (i, 0)),
            pl.BlockSpec((1, NA, DM), lambda i, e: (e[0], 0, 0)),
            pl.BlockSpec((NA, E), lambda i, e: (0, 0)),
        ],
        out_specs=pl.BlockSpec((NA, TILE), lambda i, e: (0, i)),
    )
    out_t = pl.pallas_call(
        _expert_body,
        grid_spec=grid_spec,
        out_shape=jax.ShapeDtypeStruct((NA, T), jnp.float32),
        compiler_params=pltpu.CompilerParams(
            dimension_semantics=("arbitrary",),
        ),
    )(eidx, obs, wet, bet)
    return out_t.T


# speedup vs baseline: 2.2399x; 2.2399x over previous
import jax
import jax.numpy as jnp
from jax.experimental import pallas as pl
from jax.experimental.pallas import tpu as pltpu

T = 8192
DM = 1024
E = 64
NA = 32
TILE = 512
NT = T // TILE
_BIG = 2**30

_CONTRACT_MINOR = (((1,), (1,)), ((), ()))


def _body(obs_hbm, wgt_ref, bg_ref, wet_hbm, bet_ref, out_ref,
          obs_v, w_buf, sems, wsem):
    for i in range(NT):
        pltpu.make_async_copy(
            obs_hbm.at[pl.ds(i * TILE, TILE)],
            obs_v.at[pl.ds(i * TILE, TILE)],
            sems.at[i],
        ).start()

    bgt = bg_ref[...].T
    bv = None
    for i in range(NT):
        pltpu.make_async_copy(
            obs_hbm.at[pl.ds(i * TILE, TILE)],
            obs_v.at[pl.ds(i * TILE, TILE)],
            sems.at[i],
        ).wait()
        x = obs_v[pl.ds(i * TILE, TILE), :]
        logits_t = jax.lax.dot_general(
            wgt_ref[...], x, _CONTRACT_MINOR,
            preferred_element_type=jnp.float32) + bgt
        m = jnp.max(logits_t)
        erow = jax.lax.broadcasted_iota(jnp.int32, (E, TILE), 0)
        tcol = jax.lax.broadcasted_iota(jnp.int32, (E, TILE), 1)
        flat = (i * TILE + tcol) * E + erow
        idx = jnp.min(jnp.where(logits_t == m, flat, _BIG))
        if bv is None:
            bv, bi = m, idx
        else:
            better = (m > bv) | ((m == bv) & (idx < bi))
            bv = jnp.where(better, m, bv)
            bi = jnp.where(better, idx, bi)

    e = bi % E
    pltpu.make_async_copy(wet_hbm.at[e], w_buf, wsem).start()
    cols = jax.lax.broadcasted_iota(jnp.int32, (NA, E), 1)
    b = jnp.sum(jnp.where(cols == e, bet_ref[...], 0.0),
                axis=1, keepdims=True)
    pltpu.make_async_copy(wet_hbm.at[e], w_buf, wsem).wait()

    for i in range(NT):
        x = obs_v[pl.ds(i * TILE, TILE), :]
        out_ref[:, pl.ds(i * TILE, TILE)] = jax.lax.dot_general(
            w_buf[...], x, _CONTRACT_MINOR,
            precision=jax.lax.Precision.DEFAULT,
            preferred_element_type=jnp.float32) + b


def kernel(context, obs, Wg, bg, We, be):
    del context
    wgt = Wg.T
    wet = jnp.swapaxes(We, 1, 2)
    bet = be.T
    bg2 = bg.reshape(1, E)

    out_t = pl.pallas_call(
        _body,
        in_specs=[
            pl.BlockSpec(memory_space=pltpu.MemorySpace.HBM),
            pl.BlockSpec((E, DM), lambda: (0, 0)),
            pl.BlockSpec((1, E), lambda: (0, 0)),
            pl.BlockSpec(memory_space=pltpu.MemorySpace.HBM),
            pl.BlockSpec((NA, E), lambda: (0, 0)),
        ],
        out_specs=pl.BlockSpec((NA, T), lambda: (0, 0)),
        out_shape=jax.ShapeDtypeStruct((NA, T), jnp.float32),
        scratch_shapes=[
            pltpu.VMEM((T, DM), jnp.float32),
            pltpu.VMEM((NA, DM), jnp.float32),
            pltpu.SemaphoreType.DMA((NT,)),
            pltpu.SemaphoreType.DMA,
        ],
    )(obs, wgt, bg2, wet, bet)
    return out_t.T


# speedup vs baseline: 2.4650x; 1.1005x over previous
import jax
import jax.numpy as jnp
from jax.experimental import pallas as pl
from jax.experimental.pallas import tpu as pltpu

T = 8192
DM = 1024
E = 64
NA = 32
TILE = 1024
NT = T // TILE
_BIG = 2**30

_CONTRACT_MINOR = (((1,), (1,)), ((), ()))


def _body(obs_hbm, wgt_ref, bg_ref, wet_hbm, bet_ref, out_ref,
          obs_v, w_buf, sems, wsem):
    for i in range(NT):
        pltpu.make_async_copy(
            obs_hbm.at[pl.ds(i * TILE, TILE)],
            obs_v.at[pl.ds(i * TILE, TILE)],
            sems.at[i],
        ).start()

    bgt = bg_ref[...].T
    bv = None
    for i in range(NT):
        pltpu.make_async_copy(
            obs_hbm.at[pl.ds(i * TILE, TILE)],
            obs_v.at[pl.ds(i * TILE, TILE)],
            sems.at[i],
        ).wait()
        x = obs_v[pl.ds(i * TILE, TILE), :]
        logits_t = jax.lax.dot_general(
            wgt_ref[...], x, _CONTRACT_MINOR,
            preferred_element_type=jnp.float32) + bgt
        m = jnp.max(logits_t)
        erow = jax.lax.broadcasted_iota(jnp.int32, (E, TILE), 0)
        tcol = jax.lax.broadcasted_iota(jnp.int32, (E, TILE), 1)
        flat = (i * TILE + tcol) * E + erow
        idx = jnp.min(jnp.where(logits_t == m, flat, _BIG))
        if bv is None:
            bv, bi = m, idx
        else:
            better = (m > bv) | ((m == bv) & (idx < bi))
            bv = jnp.where(better, m, bv)
            bi = jnp.where(better, idx, bi)

    e = bi % E
    pltpu.make_async_copy(wet_hbm.at[e], w_buf, wsem).start()
    cols = jax.lax.broadcasted_iota(jnp.int32, (NA, E), 1)
    b = jnp.sum(jnp.where(cols == e, bet_ref[...], 0.0),
                axis=1, keepdims=True)
    pltpu.make_async_copy(wet_hbm.at[e], w_buf, wsem).wait()

    out_ref[...] = jax.lax.dot_general(
        w_buf[...], obs_v[...], _CONTRACT_MINOR,
        preferred_element_type=jnp.float32) + b


def kernel(context, obs, Wg, bg, We, be):
    del context
    wgt = Wg.T
    wet = jnp.swapaxes(We, 1, 2)
    bet = be.T
    bg2 = bg.reshape(1, E)

    out_t = pl.pallas_call(
        _body,
        in_specs=[
            pl.BlockSpec(memory_space=pltpu.MemorySpace.HBM),
            pl.BlockSpec((E, DM), lambda: (0, 0)),
            pl.BlockSpec((1, E), lambda: (0, 0)),
            pl.BlockSpec(memory_space=pltpu.MemorySpace.HBM),
            pl.BlockSpec((NA, E), lambda: (0, 0)),
        ],
        out_specs=pl.BlockSpec((NA, T), lambda: (0, 0)),
        out_shape=jax.ShapeDtypeStruct((NA, T), jnp.float32),
        scratch_shapes=[
            pltpu.VMEM((T, DM), jnp.float32),
            pltpu.VMEM((NA, DM), jnp.float32),
            pltpu.SemaphoreType.DMA((NT,)),
            pltpu.SemaphoreType.DMA,
        ],
    )(obs, wgt, bg2, wet, bet)
    return out_t.T
